# baseline (speedup 1.0000x reference)
import functools

import jax
import jax.numpy as jnp
from jax.experimental import pallas as pl
from jax.experimental.pallas import tpu as pltpu

_B, _A, _D = 1024, 60, 5
_FA, _FB, _CONV = 128, 16, 128
_M = 32


def _tc_body(edges_ref, atoms_ref, bonds_ref, wa_ref, wb_ref, bias_ref, out_ref):
    m = _M
    r = m * _A
    edges = edges_ref[...]

    col = jax.lax.broadcasted_iota(jnp.int32, (m, _A, _A), 2)
    c = (edges[:, :, 0:1] == col).astype(jnp.float32)
    for d in range(1, _D):
        c = c + (edges[:, :, d : d + 1] == col).astype(jnp.float32)

    atoms3 = atoms_ref[...]
    g3 = jax.lax.dot_general(
        c,
        atoms3,
        dimension_numbers=(((2,), (1,)), ((0,), (0,))),
        preferred_element_type=jnp.float32,
    )
    atoms = atoms3.reshape(r, _FA)
    g = g3.reshape(r, _FA) + atoms
    ones8 = jnp.ones((m, _A, 8), dtype=jnp.float32)
    deg8 = jax.lax.dot_general(
        c,
        ones8,
        dimension_numbers=(((2,), (1,)), ((0,), (0,))),
        preferred_element_type=jnp.float32,
    )
    deg = deg8.reshape(r, 8)[:, 0:1]
    bonds = bonds_ref[...].reshape(r, _D * _FB)

    y = (
        jnp.dot(g, wa_ref[...], preferred_element_type=jnp.float32)
        + jnp.dot(bonds, wb_ref[...], preferred_element_type=jnp.float32)
        + bias_ref[...]
    )
    y = jnp.maximum(y, 0.0)

    out = jnp.zeros((r, _CONV), dtype=jnp.float32)
    for t in range(_D):
        sel = (deg == float(t + 1)).astype(jnp.float32)
        out = out + sel * y[:, t * _CONV : (t + 1) * _CONV]
    out_ref[...] = out.reshape(m, _A, _CONV)


@jax.jit
def kernel(atoms, bonds, edges, W, b):
    w_all = W.transpose(1, 0, 2).reshape(_FA + _FB, _D * _CONV)
    w_atom = w_all[:_FA]
    w_bond = jnp.tile(w_all[_FA:], (_D, 1))
    bias = b.reshape(1, _D * _CONV)
    bonds_flat = bonds.reshape(_B, _A, _D * _FB)

    grid = (_B // _M,)
    return pl.pallas_call(
        _tc_body,
        grid=grid,
        in_specs=[
            pl.BlockSpec((_M, _A, _D), lambda i: (i, 0, 0)),
            pl.BlockSpec((_M, _A, _FA), lambda i: (i, 0, 0)),
            pl.BlockSpec((_M, _A, _D * _FB), lambda i: (i, 0, 0)),
            pl.BlockSpec((_FA, _D * _CONV), lambda i: (0, 0)),
            pl.BlockSpec((_D * _FB, _D * _CONV), lambda i: (0, 0)),
            pl.BlockSpec((1, _D * _CONV), lambda i: (0, 0)),
        ],
        out_specs=pl.BlockSpec((_M, _A, _CONV), lambda i: (i, 0, 0)),
        out_shape=jax.ShapeDtypeStruct((_B, _A, _CONV), jnp.float32),
        compiler_params=pltpu.CompilerParams(
            dimension_semantics=("arbitrary",),
        ),
    )(edges, atoms, bonds_flat, w_atom, w_bond, bias)


# speedup vs baseline: 1.1190x; 1.1190x over previous
import functools

import jax
import jax.numpy as jnp
from jax.experimental import pallas as pl
from jax.experimental.pallas import tpu as pltpu

_B, _A, _D = 1024, 60, 5
_FA, _FB, _CONV = 128, 16, 128
_M = 32


def _tc_body(edges_ref, atoms_ref, bonds_ref, wa_ref, wb_ref, bias_ref, out_ref):
    m = _M
    r = m * _A
    edges = edges_ref[...]

    col = jax.lax.broadcasted_iota(jnp.int32, (m, _A, _A), 2)
    c = (edges[:, :, 0:1] == col).astype(jnp.float32)
    for d in range(1, _D):
        c = c + (edges[:, :, d : d + 1] == col).astype(jnp.float32)

    atoms3 = atoms_ref[...]
    g3 = jax.lax.dot_general(
        c,
        atoms3,
        dimension_numbers=(((2,), (1,)), ((0,), (0,))),
        preferred_element_type=jnp.float32,
    )
    atoms = atoms3.reshape(r, _FA)
    g = g3.reshape(r, _FA) + atoms
    valid = edges.reshape(r, _D) >= 0
    deg = jnp.sum(valid.astype(jnp.float32), axis=1, keepdims=True)
    bonds = bonds_ref[...].reshape(r, _D * _FB)

    y = (
        jnp.dot(g.astype(jnp.bfloat16), wa_ref[...], preferred_element_type=jnp.float32)
        + jnp.dot(bonds.astype(jnp.bfloat16), wb_ref[...], preferred_element_type=jnp.float32)
        + bias_ref[...]
    )
    y = jnp.maximum(y, 0.0)

    out = jnp.zeros((r, _CONV), dtype=jnp.float32)
    for t in range(_D):
        sel = (deg == float(t + 1)).astype(jnp.float32)
        out = out + sel * y[:, t * _CONV : (t + 1) * _CONV]
    out_ref[...] = out.reshape(m, _A, _CONV)


@jax.jit
def kernel(atoms, bonds, edges, W, b):
    w_all = W.transpose(1, 0, 2).reshape(_FA + _FB, _D * _CONV)
    w_atom = w_all[:_FA].astype(jnp.bfloat16)
    w_bond = jnp.tile(w_all[_FA:], (_D, 1)).astype(jnp.bfloat16)
    bias = b.reshape(1, _D * _CONV)
    bonds_flat = bonds.reshape(_B, _A, _D * _FB)

    grid = (_B // _M,)
    return pl.pallas_call(
        _tc_body,
        grid=grid,
        in_specs=[
            pl.BlockSpec((_M, _A, _D), lambda i: (i, 0, 0)),
            pl.BlockSpec((_M, _A, _FA), lambda i: (i, 0, 0)),
            pl.BlockSpec((_M, _A, _D * _FB), lambda i: (i, 0, 0)),
            pl.BlockSpec((_FA, _D * _CONV), lambda i: (0, 0)),
            pl.BlockSpec((_D * _FB, _D * _CONV), lambda i: (0, 0)),
            pl.BlockSpec((1, _D * _CONV), lambda i: (0, 0)),
        ],
        out_specs=pl.BlockSpec((_M, _A, _CONV), lambda i: (i, 0, 0)),
        out_shape=jax.ShapeDtypeStruct((_B, _A, _CONV), jnp.float32),
        compiler_params=pltpu.CompilerParams(
            dimension_semantics=("arbitrary",),
        ),
    )(edges, atoms, bonds_flat, w_atom, w_bond, bias)


# speedup vs baseline: 1.1282x; 1.0082x over previous
import functools

import jax
import jax.numpy as jnp
from jax.experimental import pallas as pl
from jax.experimental.pallas import tpu as pltpu

_B, _A, _D = 1024, 60, 5
_FA, _FB, _CONV = 128, 16, 128
_M = 32


def _tc_body(edges_ref, atoms_ref, bonds_ref, wa_ref, wb_ref, bias_ref, out_ref):
    m = _M
    r = m * _A
    edges = edges_ref[...]

    col = jax.lax.broadcasted_iota(jnp.int32, (m, _A, _A), 2)
    c = (edges[:, :, 0:1] == col).astype(jnp.bfloat16)
    for d in range(1, _D):
        c = c + (edges[:, :, d : d + 1] == col).astype(jnp.bfloat16)

    atoms3 = atoms_ref[...]
    g3 = jax.lax.dot_general(
        c,
        atoms3.astype(jnp.bfloat16),
        dimension_numbers=(((2,), (1,)), ((0,), (0,))),
        preferred_element_type=jnp.float32,
    )
    atoms = atoms3.reshape(r, _FA)
    g = g3.reshape(r, _FA) + atoms
    valid = edges.reshape(r, _D) >= 0
    deg = jnp.sum(valid.astype(jnp.float32), axis=1, keepdims=True)
    bonds = bonds_ref[...].reshape(r, _D * _FB)

    y = (
        jnp.dot(g.astype(jnp.bfloat16), wa_ref[...], preferred_element_type=jnp.float32)
        + jnp.dot(bonds.astype(jnp.bfloat16), wb_ref[...], preferred_element_type=jnp.float32)
        + bias_ref[...]
    )
    y = jnp.maximum(y, 0.0)

    out = jnp.zeros((r, _CONV), dtype=jnp.float32)
    for t in range(_D):
        sel = (deg == float(t + 1)).astype(jnp.float32)
        out = out + sel * y[:, t * _CONV : (t + 1) * _CONV]
    out_ref[...] = out.reshape(m, _A, _CONV)


@jax.jit
def kernel(atoms, bonds, edges, W, b):
    w_all = W.transpose(1, 0, 2).reshape(_FA + _FB, _D * _CONV)
    w_atom = w_all[:_FA].astype(jnp.bfloat16)
    w_bond = jnp.tile(w_all[_FA:], (_D, 1)).astype(jnp.bfloat16)
    bias = b.reshape(1, _D * _CONV)
    bonds_flat = bonds.reshape(_B, _A, _D * _FB)

    grid = (_B // _M,)
    return pl.pallas_call(
        _tc_body,
        grid=grid,
        in_specs=[
            pl.BlockSpec((_M, _A, _D), lambda i: (i, 0, 0)),
            pl.BlockSpec((_M, _A, _FA), lambda i: (i, 0, 0)),
            pl.BlockSpec((_M, _A, _D * _FB), lambda i: (i, 0, 0)),
            pl.BlockSpec((_FA, _D * _CONV), lambda i: (0, 0)),
            pl.BlockSpec((_D * _FB, _D * _CONV), lambda i: (0, 0)),
            pl.BlockSpec((1, _D * _CONV), lambda i: (0, 0)),
        ],
        out_specs=pl.BlockSpec((_M, _A, _CONV), lambda i: (i, 0, 0)),
        out_shape=jax.ShapeDtypeStruct((_B, _A, _CONV), jnp.float32),
        compiler_params=pltpu.CompilerParams(
            dimension_semantics=("arbitrary",),
        ),
    )(edges, atoms, bonds_flat, w_atom, w_bond, bias)
